# baseline (speedup 1.0000x reference)
import functools

import jax
import jax.numpy as jnp
from jax.experimental import pallas as pl


_ONES_COL = 16


def _prep_kernel(h_ref, w_ref, b_ref, t_ref):
    t_ref[...] = (
        jnp.dot(h_ref[...], w_ref[...], preferred_element_type=jnp.float32)
        + b_ref[...]
    )


def _pass1_kernel(a_ref, t_ref, y_ref):
    g = jnp.dot(a_ref[...], t_ref[...], preferred_element_type=jnp.float32)
    col = jax.lax.broadcasted_iota(jnp.int32, g.shape, 1)
    y_ref[...] = jnp.where(col == _ONES_COL, 1.0, jnp.maximum(g, 0.0))


def _pass2_kernel(a_ref, y_ref, w_ref, o_ref):
    u = jnp.dot(a_ref[...], y_ref[...], preferred_element_type=jnp.float32)
    o_ref[...] = jnp.dot(u, w_ref[...], preferred_element_type=jnp.float32)


@functools.partial(jax.jit, static_argnames=("interpret",))
def kernel(a, h, W0, b0, W1, b1, interpret=False):
    n = a.shape[0]
    d_in = h.shape[1]
    d_hid = W0.shape[1]
    d_out = W1.shape[1]
    p = 32

    w0_aug = jnp.zeros((d_in, p), jnp.float32).at[:, :d_hid].set(W0)
    b0_aug = jnp.zeros((1, p), jnp.float32).at[0, :d_hid].set(b0)
    w1_aug = jnp.zeros((p, d_out), jnp.float32).at[:d_hid, :].set(W1)
    w1_aug = w1_aug.at[d_hid, :].set(b1)

    t_aug = pl.pallas_call(
        _prep_kernel,
        out_shape=jax.ShapeDtypeStruct((n, p), jnp.float32),
        interpret=interpret,
    )(h, w0_aug, b0_aug)

    bm = 400
    grid = (n // bm,)

    y_aug = pl.pallas_call(
        _pass1_kernel,
        grid=grid,
        in_specs=[
            pl.BlockSpec((bm, n), lambda i: (i, 0)),
            pl.BlockSpec((n, p), lambda i: (0, 0)),
        ],
        out_specs=pl.BlockSpec((bm, p), lambda i: (i, 0)),
        out_shape=jax.ShapeDtypeStruct((n, p), jnp.float32),
        interpret=interpret,
    )(a, t_aug)

    out = pl.pallas_call(
        _pass2_kernel,
        grid=grid,
        in_specs=[
            pl.BlockSpec((bm, n), lambda i: (i, 0)),
            pl.BlockSpec((n, p), lambda i: (0, 0)),
            pl.BlockSpec((p, d_out), lambda i: (0, 0)),
        ],
        out_specs=pl.BlockSpec((bm, d_out), lambda i: (i, 0)),
        out_shape=jax.ShapeDtypeStruct((n, d_out), jnp.float32),
        interpret=interpret,
    )(a, y_aug, w1_aug)

    return out


# speedup vs baseline: 1.0564x; 1.0564x over previous
import functools

import jax
import jax.numpy as jnp
from jax.experimental import pallas as pl
from jax.experimental.pallas import tpu as pltpu

_P = 32


def _fused_kernel(a_ref, h_ref, w0_ref, b0_ref, w1_ref, o_ref, t_ref, y_ref,
                  *, bm, ones_col):
    phase = pl.program_id(0)
    i = pl.program_id(1)

    @pl.when((phase == 0) & (i == 0))
    def _prep():
        t_ref[...] = (
            jnp.dot(h_ref[...], w0_ref[...], preferred_element_type=jnp.float32)
            + b0_ref[...]
        )

    @pl.when(phase == 0)
    def _pass1():
        g = jnp.dot(a_ref[...], t_ref[...], preferred_element_type=jnp.float32)
        col = jax.lax.broadcasted_iota(jnp.int32, g.shape, 1)
        y_ref[pl.ds(i * bm, bm), :] = jnp.where(
            col == ones_col, 1.0, jnp.maximum(g, 0.0))

    @pl.when(phase == 1)
    def _pass2():
        u = jnp.dot(a_ref[...], y_ref[...], preferred_element_type=jnp.float32)
        o_ref[...] = jnp.dot(u, w1_ref[...], preferred_element_type=jnp.float32)


@functools.partial(jax.jit, static_argnames=("interpret",))
def kernel(a, h, W0, b0, W1, b1, interpret=False):
    n = a.shape[0]
    d_in = h.shape[1]
    d_hid = W0.shape[1]
    d_out = W1.shape[1]

    w0_aug = jnp.zeros((d_in, _P), jnp.float32).at[:, :d_hid].set(W0)
    b0_aug = jnp.zeros((1, _P), jnp.float32).at[0, :d_hid].set(b0)
    w1_aug = jnp.zeros((_P, d_out), jnp.float32).at[:d_hid, :].set(W1)
    w1_aug = w1_aug.at[d_hid, :].set(b1)

    bm = 400
    ni = n // bm

    out = pl.pallas_call(
        functools.partial(_fused_kernel, bm=bm, ones_col=d_hid),
        grid=(2, ni),
        in_specs=[
            pl.BlockSpec((bm, n), lambda p, i: (i, 0)),
            pl.BlockSpec((n, d_in), lambda p, i: (0, 0)),
            pl.BlockSpec((d_in, _P), lambda p, i: (0, 0)),
            pl.BlockSpec((1, _P), lambda p, i: (0, 0)),
            pl.BlockSpec((_P, d_out), lambda p, i: (0, 0)),
        ],
        out_specs=pl.BlockSpec((bm, d_out),
                               lambda p, i: (jnp.where(p == 1, i, 0), 0)),
        out_shape=jax.ShapeDtypeStruct((n, d_out), jnp.float32),
        scratch_shapes=[
            pltpu.VMEM((n, _P), jnp.float32),
            pltpu.VMEM((n, _P), jnp.float32),
        ],
        compiler_params=pltpu.CompilerParams(
            dimension_semantics=("arbitrary", "arbitrary")),
        interpret=interpret,
    )(a, h, w0_aug, b0_aug, w1_aug)

    return out
